# baseline (speedup 1.0000x reference)
import functools

import jax
import jax.numpy as jnp
from jax import lax
from jax.experimental import pallas as pl
from jax.experimental.pallas import tpu as pltpu
from jax.experimental.pallas import tpu_sc as plsc

_BATCH = 64
_SEQ_LEN = 262144
_MAX_DROP_LENGTH = 2048
_WIN = _MAX_DROP_LENGTH + 16
_NW = 32
_ROWS_PER_W = _BATCH // _NW


def _drop_bounds(batch, seq_len):
    rkey = jax.random.key(42)
    k_start, k_len = jax.random.split(rkey)
    drop_start = jax.random.randint(k_start, (batch,), 0, seq_len // 2)
    drop_len = jax.random.randint(k_len, (batch,), 0, _MAX_DROP_LENGTH)
    drop_end = jnp.minimum(drop_start + drop_len, seq_len)
    return drop_start.astype(jnp.int32), drop_end.astype(jnp.int32)


def _scalar_at(vmem_ref, i):
    base = (i // 16) * 16
    v = vmem_ref[pl.ds(base, 16)]
    lane = lax.broadcasted_iota(jnp.int32, (16,), 0)
    return jnp.max(jnp.where(lane == i % 16, v, 0), axis=0)


def _aligned_scalar_at(vmem_ref, i):
    return pl.multiple_of(_scalar_at(vmem_ref, i), 16)


def _sc_body(in_hbm, sa_hbm, rs_hbm, re_hbm, out_hbm,
             sa_v, rs_v, re_v, wbuf, sem_copy, sem_win, sem_wb):
    wid = lax.axis_index("s") * 2 + lax.axis_index("c")

    pltpu.sync_copy(sa_hbm, sa_v)
    pltpu.sync_copy(rs_hbm, rs_v)
    pltpu.sync_copy(re_hbm, re_v)

    copies = []
    for k in range(_ROWS_PER_W):
        r = wid * _ROWS_PER_W + k
        copies.append(
            pltpu.async_copy(in_hbm.at[r], out_hbm.at[r], sem_copy[k]))
        sa = _aligned_scalar_at(sa_v, r)
        pltpu.async_copy(
            in_hbm.at[r, pl.ds(sa, _WIN)], wbuf.at[k], sem_win[k])

    for k in range(_ROWS_PER_W):
        r = wid * _ROWS_PER_W + k
        sa = _aligned_scalar_at(sa_v, r)
        rs = _scalar_at(rs_v, r)
        re = _scalar_at(re_v, r)
        pltpu.make_async_copy(
            in_hbm.at[r, pl.ds(sa, _WIN)], wbuf.at[k], sem_win[k]).wait()

        def chunk(j, _, k=k, rs=rs, re=re):
            col = j * 16 + lax.broadcasted_iota(jnp.int32, (16,), 0)
            val = wbuf[k, pl.ds(j * 16, 16)]
            keep = (col >= rs) & (col < re)
            wbuf[k, pl.ds(j * 16, 16)] = jnp.where(keep, 0.0, val)
            return 0

        lax.fori_loop(0, _WIN // 16, chunk, 0)

        copies[k].wait()
        pltpu.async_copy(
            wbuf.at[k], out_hbm.at[r, pl.ds(sa, _WIN)], sem_wb[k])

    for k in range(_ROWS_PER_W):
        r = wid * _ROWS_PER_W + k
        sa = _aligned_scalar_at(sa_v, r)
        pltpu.make_async_copy(
            wbuf.at[k], out_hbm.at[r, pl.ds(sa, _WIN)], sem_wb[k]).wait()


def kernel(waveform):
    batch, seq_len = waveform.shape
    s, e = _drop_bounds(batch, seq_len)
    sa = (s // 16) * 16
    rs = s - sa
    re = e - sa

    mesh = plsc.VectorSubcoreMesh(core_axis_name="c", subcore_axis_name="s")
    run = pl.kernel(
        _sc_body,
        out_type=jax.ShapeDtypeStruct((batch, seq_len), waveform.dtype),
        mesh=mesh,
        compiler_params=pltpu.CompilerParams(use_tc_tiling_on_sc=False,
                                             needs_layout_passes=False),
        scratch_types=[
            pltpu.VMEM((batch,), jnp.int32),
            pltpu.VMEM((batch,), jnp.int32),
            pltpu.VMEM((batch,), jnp.int32),
            pltpu.VMEM((_ROWS_PER_W, _WIN), jnp.float32),
            [pltpu.SemaphoreType.DMA] * _ROWS_PER_W,
            [pltpu.SemaphoreType.DMA] * _ROWS_PER_W,
            [pltpu.SemaphoreType.DMA] * _ROWS_PER_W,
        ],
    )
    return run(waveform, sa, rs, re)


# speedup vs baseline: 11.2042x; 11.2042x over previous
import jax
import jax.numpy as jnp
from jax import lax
from jax.experimental import pallas as pl
from jax.experimental.pallas import tpu as pltpu
from jax.experimental.pallas import tpu_sc as plsc

_BATCH = 64
_SEQ_LEN = 262144
_MAX_DROP_LENGTH = 2048
_NW = 32
_ROWS_PER_W = _BATCH // _NW
_SPAN = _ROWS_PER_W * _SEQ_LEN
_CH = 32768
_NCH = _SPAN // _CH
_NBUF = 3


def _drop_bounds(batch, seq_len):
    rkey = jax.random.key(42)
    k_start, k_len = jax.random.split(rkey)
    drop_start = jax.random.randint(k_start, (batch,), 0, seq_len // 2)
    drop_len = jax.random.randint(k_len, (batch,), 0, _MAX_DROP_LENGTH)
    drop_end = jnp.minimum(drop_start + drop_len, seq_len)
    return drop_start.astype(jnp.int32), drop_end.astype(jnp.int32)


def _scalar_at(vmem_ref, i):
    base = (i // 16) * 16
    v = vmem_ref[pl.ds(base, 16)]
    lane = lax.broadcasted_iota(jnp.int32, (16,), 0)
    return jnp.max(jnp.where(lane == i % 16, v, 0), axis=0)


def _sc_body(in_hbm, s_hbm, e_hbm, out_hbm, s_v, e_v, bufs, sem_in, sem_out):
    wid = lax.axis_index("s") * 2 + lax.axis_index("c")
    base = wid * _SPAN

    pltpu.sync_copy(s_hbm, s_v)
    pltpu.sync_copy(e_hbm, e_v)
    row0 = wid * _ROWS_PER_W
    ws = [_scalar_at(s_v, row0 + k) for k in range(_ROWS_PER_W)]
    we = [_scalar_at(e_v, row0 + k) for k in range(_ROWS_PER_W)]

    def start_in(c):
        return pltpu.async_copy(
            in_hbm.at[pl.ds(base + c * _CH, _CH)],
            bufs.at[c % _NBUF], sem_in[c % _NBUF])

    def start_out(c):
        return pltpu.async_copy(
            bufs.at[c % _NBUF],
            out_hbm.at[pl.ds(base + c * _CH, _CH)], sem_out[c % _NBUF])

    def wait_in(c):
        pltpu.make_async_copy(
            in_hbm.at[pl.ds(base + c * _CH, _CH)],
            bufs.at[c % _NBUF], sem_in[c % _NBUF]).wait()

    def wait_out(c):
        pltpu.make_async_copy(
            bufs.at[c % _NBUF],
            out_hbm.at[pl.ds(base + c * _CH, _CH)], sem_out[c % _NBUF]).wait()

    def zero_window(c, p, s, e):
        c0 = base + c * _CH
        lo = jnp.maximum(s, c0)
        hi = jnp.minimum(e, c0 + _CH)

        def granule(g, _):
            off = pl.multiple_of(g * 16 - c0, 16)
            col = g * 16 + lax.broadcasted_iota(jnp.int32, (16,), 0)
            val = bufs[p, pl.ds(off, 16)]
            drop = (col >= s) & (col < e)
            bufs[p, pl.ds(off, 16)] = jnp.where(drop, 0.0, val)
            return 0

        lax.fori_loop(lo // 16, (hi + 15) // 16, granule, 0)

    for c in range(min(_NBUF, _NCH)):
        start_in(c)
    for c in range(_NCH):
        p = c % _NBUF
        wait_in(c)
        for k in range(_ROWS_PER_W):
            zero_window(c, p, ws[k], we[k])
        start_out(c)
        pre = c - _NBUF + 1
        if pre >= 0 and pre + _NBUF < _NCH:
            wait_out(pre)
            start_in(pre + _NBUF)
    for c in range(max(_NCH - _NBUF, 0), _NCH):
        wait_out(c)


def kernel(waveform):
    batch, seq_len = waveform.shape
    s, e = _drop_bounds(batch, seq_len)
    row_base = jnp.arange(batch, dtype=jnp.int32) * seq_len
    s_flat = row_base + s
    e_flat = row_base + e

    mesh = plsc.VectorSubcoreMesh(core_axis_name="c", subcore_axis_name="s")
    run = pl.kernel(
        _sc_body,
        out_type=jax.ShapeDtypeStruct((batch * seq_len,), waveform.dtype),
        mesh=mesh,
        compiler_params=pltpu.CompilerParams(use_tc_tiling_on_sc=False,
                                             needs_layout_passes=False),
        scratch_types=[
            pltpu.VMEM((_BATCH,), jnp.int32),
            pltpu.VMEM((_BATCH,), jnp.int32),
            pltpu.VMEM((_NBUF, _CH), jnp.float32),
            [pltpu.SemaphoreType.DMA] * _NBUF,
            [pltpu.SemaphoreType.DMA] * _NBUF,
        ],
    )
    out_flat = run(waveform.reshape(-1), s_flat, e_flat)
    return out_flat.reshape(batch, seq_len)


# speedup vs baseline: 15.0388x; 1.3422x over previous
import jax
import jax.numpy as jnp
from jax import lax
from jax.experimental import pallas as pl
from jax.experimental.pallas import tpu as pltpu
from jax.experimental.pallas import tpu_sc as plsc

_BATCH = 64
_SEQ_LEN = 262144
_MAX_DROP_LENGTH = 2048
_WIN = _MAX_DROP_LENGTH + 16
_NW = 32
_ROWS_PER_W = _BATCH // _NW


def _drop_bounds(batch, seq_len):
    rkey = jax.random.key(42)
    k_start, k_len = jax.random.split(rkey)
    drop_start = jax.random.randint(k_start, (batch,), 0, seq_len // 2)
    drop_len = jax.random.randint(k_len, (batch,), 0, _MAX_DROP_LENGTH)
    drop_end = jnp.minimum(drop_start + drop_len, seq_len)
    return drop_start.astype(jnp.int32), drop_end.astype(jnp.int32)


def _scalar_at(vmem_ref, i):
    base = (i // 16) * 16
    v = vmem_ref[pl.ds(base, 16)]
    lane = lax.broadcasted_iota(jnp.int32, (16,), 0)
    return jnp.max(jnp.where(lane == i % 16, v, 0), axis=0)


def _sc_fix_body(sa_hbm, s_hbm, e_hbm, buf_hbm,
                 sa_v, s_v, e_v, wbuf, sem_in, sem_out):
    wid = lax.axis_index("s") * 2 + lax.axis_index("c")
    row0 = wid * _ROWS_PER_W

    pltpu.sync_copy(sa_hbm, sa_v)
    pltpu.sync_copy(s_hbm, s_v)
    pltpu.sync_copy(e_hbm, e_v)

    sas = [pl.multiple_of(_scalar_at(sa_v, row0 + k), 16)
           for k in range(_ROWS_PER_W)]
    for k in range(_ROWS_PER_W):
        pltpu.async_copy(
            buf_hbm.at[pl.ds(sas[k], _WIN)], wbuf.at[k], sem_in[k])

    for k in range(_ROWS_PER_W):
        r = row0 + k
        s = _scalar_at(s_v, r)
        e = _scalar_at(e_v, r)
        pltpu.make_async_copy(
            buf_hbm.at[pl.ds(sas[k], _WIN)], wbuf.at[k], sem_in[k]).wait()

        def granule(g, _, k=k, s=s, e=e, sa=sas[k]):
            off = pl.multiple_of(g * 16 - sa, 16)
            col = g * 16 + lax.broadcasted_iota(jnp.int32, (16,), 0)
            val = wbuf[k, pl.ds(off, 16)]
            drop = (col >= s) & (col < e)
            wbuf[k, pl.ds(off, 16)] = jnp.where(drop, 0.0, val)
            return 0

        lax.fori_loop(s // 16, (e + 15) // 16, granule, 0)
        pltpu.async_copy(
            wbuf.at[k], buf_hbm.at[pl.ds(sas[k], _WIN)], sem_out[k])

    for k in range(_ROWS_PER_W):
        pltpu.make_async_copy(
            wbuf.at[k], buf_hbm.at[pl.ds(sas[k], _WIN)], sem_out[k]).wait()


def kernel(waveform):
    batch, seq_len = waveform.shape
    s, e = _drop_bounds(batch, seq_len)
    row_base = jnp.arange(batch, dtype=jnp.int32) * seq_len
    sa_flat = row_base + (s // 16) * 16
    s_flat = row_base + s
    e_flat = row_base + e

    mesh = plsc.VectorSubcoreMesh(core_axis_name="c", subcore_axis_name="s")
    run = pl.kernel(
        _sc_fix_body,
        mesh=mesh,
        compiler_params=pltpu.CompilerParams(use_tc_tiling_on_sc=False,
                                             needs_layout_passes=False),
        scratch_types=[
            pltpu.VMEM((_BATCH,), jnp.int32),
            pltpu.VMEM((_BATCH,), jnp.int32),
            pltpu.VMEM((_BATCH,), jnp.int32),
            pltpu.VMEM((_ROWS_PER_W, _WIN), jnp.float32),
            [pltpu.SemaphoreType.DMA] * _ROWS_PER_W,
            [pltpu.SemaphoreType.DMA] * _ROWS_PER_W,
        ],
    )
    buf = jax.new_ref(waveform.reshape(-1))
    run(sa_flat, s_flat, e_flat, buf)
    return buf[...].reshape(batch, seq_len)
